# Initial kernel scaffold; baseline (speedup 1.0000x reference)
import jax
import jax.numpy as jnp
from jax.experimental import pallas as pl


def kernel(act_score, inter_score, n_person, knowledge, params):
    raise NotImplementedError("write your pallas kernel here")


# speedup vs baseline: 39.5926x; 39.5926x over previous
import numpy as np
import jax
import jax.numpy as jnp
from jax import lax
from itertools import combinations
from jax.experimental import pallas as pl

_N = 24
_NPAIR = _N * (_N - 1) // 2
_L = _N + _NPAIR
_NR = _NPAIR + (_N * (_N - 1) * (_N - 2)) // 6
_LP, _RP = 384, 2304
_NLAYERS = 11


def _build_consts():
    pidx = {c: i for i, c in enumerate(combinations(range(_N), 2))}
    C = []
    for (u, v) in combinations(range(_N), 2):
        C.append([u, v, _N + pidx[(u, v)]])
    for (i, j, k) in combinations(range(_N), 3):
        C.append([_N + pidx[(i, j)], _N + pidx[(i, k)], _N + pidx[(j, k)]])
    C = np.array(C, np.int64)

    OH = np.zeros((3, _RP, _LP), np.float32)
    for jj in range(3):
        OH[jj, np.arange(_NR), C[:, jj]] = 1.0

    AEmb = np.zeros((_LP, _N), np.float32)
    AEmb[np.arange(_N), np.arange(_N)] = 1.0
    ZEmb = np.zeros((_LP, _N * (_N - 1)), np.float32)
    for (u, v) in combinations(range(_N), 2):
        ZEmb[_N + pidx[(u, v)], u * (_N - 1) + v - 1] = 1.0

    PSel = np.zeros((576, _LP), np.float32)
    PRC = np.zeros((576, _N), np.float32)
    RowSum = np.zeros((_N, 576), np.float32)
    q = 0
    for i in range(_N):
        for j in range(_N):
            if i == j:
                continue
            PSel[q, _N + pidx[(min(i, j), max(i, j))]] = 1.0
            PRC[q, i] += 1.0
            PRC[q, j] += 1.0
            q += 1
    for i in range(_N):
        RowSum[i, i * (_N - 1):(i + 1) * (_N - 1)] = 1.0
    return OH, AEmb, ZEmb, PSel, PRC, RowSum


_OH, _AEMB, _ZEMB, _PSEL, _PRC, _ROWSUM = _build_consts()
_MULT = (1.0, 1.0, 21.0)


def _ln_relu(x, g, b):
    mu = jnp.mean(x, axis=-1, keepdims=True)
    xc = x - mu
    var = jnp.mean(xc * xc, axis=-1, keepdims=True)
    return jax.nn.relu(xc * jax.lax.rsqrt(var + 1e-5) * g + b)


def _body(refs):
    (act_ref, inter_ref, know_ref,
     oh0_ref, oh1_ref, oh2_ref, aemb_ref, zemb_ref,
     psel_ref, prc_ref, rowsum_ref,
     aw1, ab1, aw2, ab2, alng, alnb,
     iw1, ib1, iw2, ib2, ilng, ilnb,
     fw1, fb1, fw2, fb2, icw, icb,
     ewA, ewB, eb_ref, lam_ref,
     fgnn_refs, oa_ref, oi_ref) = refs

    OH0, OH1, OH2 = oh0_ref[...], oh1_ref[...], oh2_ref[...]

    a = jax.nn.relu(act_ref[...] @ aw1[...] + ab1[...])
    a = a @ aw2[...] + ab2[...]
    a = _ln_relu(a, alng[...], alnb[...])

    s = jax.nn.relu(inter_ref[...] @ iw1[...] + ib1[...])
    s = s @ iw2[...] + ib2[...]
    s = _ln_relu(s, ilng[...], ilnb[...])

    Lf = aemb_ref[...] @ a + zemb_ref[...] @ s
    Rf = (OH0 @ Lf + OH1 @ Lf + OH2 @ Lf) * (1.0 / 3.0)

    eb = eb_ref[...]
    PA_L, PB_L = Lf @ ewA[...], Lf @ ewB[...]
    PA_R, PB_R = Rf @ ewA[...], Rf @ ewB[...]
    WR = (jax.nn.relu(PA_R + OH0 @ PB_L + eb),
          jax.nn.relu(PA_R + OH1 @ PB_L + eb),
          jax.nn.relu(PA_R + OH2 @ PB_L + eb))
    WL = (jax.nn.relu(OH0 @ PA_L + PB_R + eb),
          jax.nn.relu(OH1 @ PA_L + PB_R + eb),
          jax.nn.relu(OH2 @ PA_L + PB_R + eb))

    hL, hR = Lf, Rf
    for li in range(_NLAYERS):
        ws, wm, wer, b = fgnn_refs[4 * li:4 * li + 4]
        we = wer[...]
        D_RL = jnp.zeros((_RP, _LP), jnp.float32)
        D_LT = jnp.zeros((_RP, _LP), jnp.float32)
        for jj, OHj in enumerate((OH0, OH1, OH2)):
            vr = jnp.sum(WR[jj] * we, axis=1, keepdims=True) * (_MULT[jj] / 23.0)
            vl = jnp.sum(WL[jj] * we, axis=1, keepdims=True) * (1.0 / 23.0)
            D_RL = D_RL + jax.nn.relu(vr) * OHj
            D_LT = D_LT + jax.nn.relu(vl) * OHj
        msgR = D_RL @ hL
        msgL = lax.dot_general(D_LT, hR, (((0,), (0,)), ((), ())))
        bb = b[...]
        hL = jax.nn.relu(hL @ ws[...] + msgL @ wm[...] + bb)
        hR = jax.nn.relu(hR @ ws[...] + msgR @ wm[...] + bb)

    actn = hL[0:_N, :]
    act_out = jax.nn.relu((a + actn) @ fw1[...] + fb1[...]) @ fw2[...] + fb2[...]
    inter_out = (psel_ref[...] @ hL) @ icw[...] + icb[...]

    K = know_ref[...]
    lh = lam_ref[0, 0]
    lg = lam_ref[0, 1]
    act, inter = act_out, inter_out
    for _ in range(3):
        qa = jax.nn.softmax(act, axis=-1)
        qi = jax.nn.softmax(inter, axis=-1)
        Qs = rowsum_ref[...] @ qi
        act = act_out + lh * lax.dot_general(Qs, K, (((1,), (1,)), ((), ())))
        inter = inter_out + lg * (prc_ref[...] @ (qa @ K))
    oa_ref[...] = act
    oi_ref[...] = inter[0:552, :]


def _run(act_score, inter_score, knowledge, flat_params):
    n_in = 11 + 22 + 4 * _NLAYERS

    def body(*refs):
        _body((*refs[0:11],
               *refs[11:33],
               refs[33:33 + 4 * _NLAYERS],
               refs[n_in], refs[n_in + 1]))

    consts = (jnp.asarray(_OH[0]), jnp.asarray(_OH[1]), jnp.asarray(_OH[2]),
              jnp.asarray(_AEMB), jnp.asarray(_ZEMB),
              jnp.asarray(_PSEL), jnp.asarray(_PRC), jnp.asarray(_ROWSUM))
    return pl.pallas_call(
        body,
        out_shape=[jax.ShapeDtypeStruct((_N, 32), jnp.float32),
                   jax.ShapeDtypeStruct((552, 2), jnp.float32)],
    )(act_score, inter_score, knowledge, *consts, *flat_params)


def kernel(act_score, inter_score, n_person, knowledge, params):
    p = params
    r2 = lambda x: x.reshape(1, -1)
    flat = [p['aff_w1'], r2(p['aff_b1']), p['aff_w2'], r2(p['aff_b2']),
            r2(p['aff_ln_g']), r2(p['aff_ln_b']),
            p['iff_w1'], r2(p['iff_b1']), p['iff_w2'], r2(p['iff_b2']),
            r2(p['iff_ln_g']), r2(p['iff_ln_b']),
            p['afc_w1'], r2(p['afc_b1']), p['afc_w2'], r2(p['afc_b2']),
            p['ifc_w'], r2(p['ifc_b']),
            p['edge_w'][:128, :], p['edge_w'][128:, :], r2(p['edge_b']),
            jnp.stack([p['lambda_h'][0], p['lambda_g'][0]]).reshape(1, 2)]
    for (ws, wm, we, b) in p['fgnn']:
        flat.extend([ws, wm, r2(we), r2(b)])
    act, inter = _run(act_score, inter_score, knowledge, flat)
    return act, inter
